# baseline (speedup 1.0000x reference)
import functools
import math

import jax
import jax.numpy as jnp
from jax import lax
from jax.experimental import pallas as pl
from jax.experimental.pallas import tpu as pltpu
from jax.experimental.pallas import tpu_sc as plsc

_L = 16


def _make_impl(B, NF, CARD, D, NC, NS):
    NW = NC * NS
    R = B * NF
    assert R % NW == 0
    per_w = R // NW
    unit = (NF * 8) // math.gcd(NF, 8)
    assert per_w % unit == 0
    n_units = per_w // unit
    chunk_units = n_units
    while chunk_units * unit * D * 4 > 220 * 1024:
        assert chunk_units % 2 == 0
        chunk_units //= 2
    CHUNK = chunk_units * unit
    NCHUNK = per_w // CHUNK
    assert D % _L == 0
    DV = D // _L

    mesh = plsc.VectorSubcoreMesh(core_axis_name="c", subcore_axis_name="s")

    @functools.partial(
        pl.kernel,
        mesh=mesh,
        out_type=jax.ShapeDtypeStruct((R, D), jnp.float32),
        compiler_params=pltpu.CompilerParams(use_tc_tiling_on_sc=False),
        scratch_types=[
            pltpu.VMEM((CHUNK,), jnp.int32),
            pltpu.VMEM((CHUNK,), jnp.int32),
            pltpu.VMEM((NF, D), jnp.float32),
            pltpu.VMEM((CHUNK, D), jnp.float32),
            pltpu.SemaphoreType.DMA,
        ],
    )
    def k(xf_hbm, tf_hbm, bias_hbm, offs_hbm, out_hbm,
          idx_v, offs_v, bias_v, rows_v, sem):
        wid = lax.axis_index("s") * NC + lax.axis_index("c")
        base = wid * per_w

        pltpu.sync_copy(bias_hbm, bias_v)
        pltpu.sync_copy(offs_hbm, offs_v)

        def add_offs(v, _):
            sl = pl.ds(v * _L, _L)
            idx_v[sl] = idx_v[sl] + offs_v[sl]
            return _

        def add_bias(g, _):
            r0 = g * NF
            for r in range(NF):
                for h in range(DV):
                    sl = pl.ds(h * _L, _L)
                    rows_v[r0 + r, sl] = rows_v[r0 + r, sl] + bias_v[r, sl]
            return _

        for c in range(NCHUNK):
            row0 = base + c * CHUNK
            pltpu.sync_copy(xf_hbm.at[pl.ds(row0, CHUNK)], idx_v)
            lax.fori_loop(0, CHUNK // _L, add_offs, None)
            pltpu.async_copy(tf_hbm.at[idx_v], rows_v, sem).wait()
            lax.fori_loop(0, CHUNK // NF, add_bias, None)
            pltpu.sync_copy(rows_v, out_hbm.at[pl.ds(row0, CHUNK)])

    return k, CHUNK


def kernel(x, tables, bias):
    B, NF = x.shape
    NF2, CARD, D = tables.shape
    assert NF2 == NF
    info = plsc.get_sparse_core_info()
    NC, NS = info.num_cores, info.num_subcores

    impl, CHUNK = _make_impl(B, NF, CARD, D, NC, NS)
    xf = x.astype(jnp.int32).reshape(-1)
    tf = tables.reshape(NF * CARD, D)
    offs = (jnp.arange(CHUNK, dtype=jnp.int32) % NF) * CARD
    out = impl(xf, tf, bias, offs)
    return out.reshape(B, NF, D)


# speedup vs baseline: 1.0004x; 1.0004x over previous
import functools
import math

import jax
import jax.numpy as jnp
from jax import lax
from jax.experimental import pallas as pl
from jax.experimental.pallas import tpu as pltpu
from jax.experimental.pallas import tpu_sc as plsc

_L = 16


def _make_impl(B, NF, CARD, D, NC, NS):
    NW = NC * NS
    R = B * NF
    assert R % NW == 0
    per_w = R // NW
    unit = (NF * 8) // math.gcd(NF, 8)
    assert per_w % unit == 0
    n_units = per_w // unit
    chunk_units = n_units
    while chunk_units * unit * D * 4 > 220 * 1024:
        assert chunk_units % 2 == 0
        chunk_units //= 2
    CHUNK = chunk_units * unit
    NCHUNK = per_w // CHUNK
    assert D % _L == 0
    DV = D // _L

    mesh = plsc.VectorSubcoreMesh(core_axis_name="c", subcore_axis_name="s")

    @functools.partial(
        pl.kernel,
        mesh=mesh,
        out_type=jax.ShapeDtypeStruct((R, D), jnp.float32),
        compiler_params=pltpu.CompilerParams(use_tc_tiling_on_sc=False),
        scratch_types=[
            pltpu.VMEM((CHUNK,), jnp.int32),
            pltpu.VMEM((CHUNK,), jnp.int32),
            pltpu.VMEM((NF, D), jnp.float32),
            pltpu.VMEM((CHUNK, D), jnp.float32),
            pltpu.SemaphoreType.DMA,
        ],
    )
    def k(xf_hbm, tf_hbm, bias_hbm, offs_hbm, out_hbm,
          idx_v, offs_v, bias_v, rows_v, sem):
        wid = lax.axis_index("s") * NC + lax.axis_index("c")
        base = wid * per_w

        pltpu.sync_copy(bias_hbm, bias_v)
        pltpu.sync_copy(offs_hbm, offs_v)

        def add_offs(v, _):
            sl = pl.ds(v * _L, _L)
            idx_v[sl] = idx_v[sl] + offs_v[sl]
            return _

        def add_bias(g, _):
            r0 = g * NF
            for r in range(NF):
                for h in range(DV):
                    sl = pl.ds(h * _L, _L)
                    rows_v[r0 + r, sl] = rows_v[r0 + r, sl] + bias_v[r, sl]
            return _

        for c in range(NCHUNK):
            row0 = base + c * CHUNK
            pltpu.sync_copy(xf_hbm.at[pl.ds(row0, CHUNK)], idx_v)
            lax.fori_loop(0, CHUNK // _L, add_offs, None)
            pltpu.async_copy(tf_hbm.at[idx_v], rows_v, sem).wait()
            lax.fori_loop(0, CHUNK // NF, add_bias, None)
            pltpu.sync_copy(rows_v, out_hbm.at[pl.ds(row0, CHUNK)])

    return k, CHUNK


def kernel(x, tables, bias):
    B, NF = x.shape
    NF2, CARD, D = tables.shape
    assert NF2 == NF
    info = plsc.get_sparse_core_info()
    NC, NS = info.num_cores, info.num_subcores

    impl, CHUNK = _make_impl(B, NF, CARD, D, NC, NS)
    xf = x.astype(jnp.int32).reshape(-1)
    offs = (jnp.arange(CHUNK, dtype=jnp.int32) % NF) * CARD
    t2 = tables.reshape(NF * CARD, D)
    out = impl(xf, t2, bias, offs)
    return out.reshape(B, NF, D)


# speedup vs baseline: 2.6159x; 2.6149x over previous
import functools

import jax
import jax.numpy as jnp
from jax import lax
from jax.experimental import pallas as pl
from jax.experimental.pallas import tpu as pltpu
from jax.experimental.pallas import tpu_sc as plsc

_L = 16


def _make_impl(B, NF, CARD, D, NC, NS):
    assert NF % NC == 0
    F_PER_CORE = NF // NC
    BW = B // NS
    assert B % NS == 0 and BW % 128 == 0
    assert D % 8 == 0
    NT = D // 8
    C_MAIN = CARD // 128 * 128
    C_TAIL = CARD - C_MAIN
    CW = (C_MAIN // 128 + NS - 1) // NS * 128
    CW_LAST = C_MAIN - (NS - 1) * CW
    assert CW_LAST > 0 and CW_LAST % 128 == 0
    SW = C_MAIN + (128 if C_TAIL else 0)
    VW = max(CW, CW_LAST + (128 if C_TAIL else 0))

    mesh = plsc.VectorSubcoreMesh(core_axis_name="c", subcore_axis_name="s")

    @functools.partial(
        pl.kernel,
        mesh=mesh,
        out_type=jax.ShapeDtypeStruct((NF, D, B), jnp.float32),
        compiler_params=pltpu.CompilerParams(use_tc_tiling_on_sc=True),
        scratch_types=[
            pltpu.VMEM((8, VW), jnp.float32),
            pltpu.VMEM((BW,), jnp.int32),
            pltpu.VMEM((4 * BW,), jnp.int32),
            pltpu.VMEM((4 * BW,), jnp.float32),
            pltpu.VMEM((8, BW), jnp.float32),
            pltpu.VMEM((NF * D * _L,), jnp.float32),
            pltpu.VMEM_SHARED((4 * SW,), jnp.float32),
            pltpu.SemaphoreType.DMA,
        ],
    )
    def k(xf_hbm, tt_hbm, tail_hbm, biasf_hbm, out_hbm,
          vm_slab, vm_c, vm_gidx, vm_flat, vm_blk, vm_bias, sh_slab, sem):
        cid = lax.axis_index("c")
        sid = lax.axis_index("s")
        c0 = sid * CW

        pltpu.sync_copy(biasf_hbm, vm_bias)

        def feature_body(fi, _):
            f = cid * F_PER_CORE + fi
            pltpu.sync_copy(xf_hbm.at[pl.ds(f * B + sid * BW, BW)], vm_c)

            def gidx_body(v, _):
                cvec = vm_c[pl.ds(v * _L, _L)]
                for r in range(4):
                    vm_gidx[pl.ds(r * BW + v * _L, _L)] = cvec + r * SW
                return _
            lax.fori_loop(0, BW // _L, gidx_body, None)

            for t in range(NT):
                @pl.when(sid < NS - 1)
                def _stage_main():
                    pltpu.sync_copy(
                        tt_hbm.at[f, pl.ds(8 * t, 8), pl.ds(c0, CW)],
                        vm_slab.at[:, pl.ds(0, CW)])

                @pl.when(sid == NS - 1)
                def _stage_last():
                    pltpu.sync_copy(
                        tt_hbm.at[f, pl.ds(8 * t, 8), pl.ds(c0, CW_LAST)],
                        vm_slab.at[:, pl.ds(0, CW_LAST)])
                    if C_TAIL:
                        pltpu.sync_copy(
                            tail_hbm.at[f, pl.ds(8 * t, 8)],
                            vm_slab.at[:, pl.ds(CW_LAST, 128)])

                for h in range(2):
                    for r in range(4):
                        @pl.when(sid < NS - 1)
                        def _pub_main(r=r, h=h):
                            pltpu.sync_copy(
                                vm_slab.at[4 * h + r, pl.ds(0, CW)],
                                sh_slab.at[pl.ds(r * SW + c0, CW)])

                        @pl.when(sid == NS - 1)
                        def _pub_last(r=r, h=h):
                            w = CW_LAST + (128 if C_TAIL else 0)
                            pltpu.sync_copy(
                                vm_slab.at[4 * h + r, pl.ds(0, w)],
                                sh_slab.at[pl.ds(r * SW + c0, w)])
                    plsc.subcore_barrier()

                    pltpu.async_copy(sh_slab.at[vm_gidx], vm_flat, sem).wait()

                    for r in range(4):
                        bvec = vm_bias[
                            pl.ds((f * D + 8 * t + 4 * h + r) * _L, _L)]

                        def row_body(v, _, r=r, h=h, bvec=bvec):
                            sl = pl.ds(v * _L, _L)
                            vm_blk[4 * h + r, sl] = (
                                vm_flat[pl.ds(r * BW + v * _L, _L)] + bvec)
                            return _
                        lax.fori_loop(0, BW // _L, row_body, None)
                    plsc.subcore_barrier()

                pltpu.sync_copy(
                    vm_blk,
                    out_hbm.at[f, pl.ds(8 * t, 8), pl.ds(sid * BW, BW)])
            return _

        lax.fori_loop(0, F_PER_CORE, feature_body, None)

    return k, C_MAIN, C_TAIL


def kernel(x, tables, bias):
    B, NF = x.shape
    NF2, CARD, D = tables.shape
    assert NF2 == NF
    info = plsc.get_sparse_core_info()
    NC, NS = info.num_cores, info.num_subcores

    impl, C_MAIN, C_TAIL = _make_impl(B, NF, CARD, D, NC, NS)
    xf = x.astype(jnp.int32).T.reshape(-1)
    tt = jnp.transpose(tables, (0, 2, 1))
    tail = jnp.transpose(tables[:, C_MAIN:, :], (0, 2, 1)) if C_TAIL \
        else jnp.zeros((NF, D, 0), tables.dtype)
    tail = jnp.pad(tail, ((0, 0), (0, 0), (0, 128 - tail.shape[2])))
    biasf = jnp.repeat(bias.reshape(-1)[:, None], 16, axis=1).reshape(-1)
    out3 = impl(xf, tt, tail, biasf)
    return jnp.transpose(out3, (2, 0, 1))


# speedup vs baseline: 3.5534x; 1.3584x over previous
import functools

import jax
import jax.numpy as jnp
from jax import lax
from jax.experimental import pallas as pl
from jax.experimental.pallas import tpu as pltpu
from jax.experimental.pallas import tpu_sc as plsc

_L = 16


def _make_impl(B, NF, CARD, D, NC, NS):
    assert NF % NC == 0
    FPC = NF // NC
    BW = B // NS
    assert B % NS == 0 and BW % 128 == 0
    assert D % 8 == 0
    NT = D // 8
    C_MAIN = CARD // 128 * 128
    C_TAIL = CARD - C_MAIN
    CW = (C_MAIN // 128 + NS - 1) // NS * 128
    CW_LAST = C_MAIN - (NS - 1) * CW
    assert CW_LAST > 0 and CW_LAST % 128 == 0
    WL = CW_LAST + (128 if C_TAIL else 0)
    SW = C_MAIN + (128 if C_TAIL else 0)
    VW = max(CW, WL)

    mesh = plsc.VectorSubcoreMesh(core_axis_name="c", subcore_axis_name="s")

    @functools.partial(
        pl.kernel,
        mesh=mesh,
        out_type=jax.ShapeDtypeStruct((NF, D, B), jnp.float32),
        compiler_params=pltpu.CompilerParams(use_tc_tiling_on_sc=True),
        scratch_types=[
            pltpu.VMEM((8, VW), jnp.float32),
            pltpu.VMEM((BW,), jnp.int32),
            pltpu.VMEM((4 * BW,), jnp.int32),
            pltpu.VMEM((4 * BW,), jnp.float32),
            pltpu.VMEM((8, BW), jnp.float32),
            pltpu.VMEM((FPC * D * _L,), jnp.float32),
            pltpu.VMEM_SHARED((4 * SW,), jnp.float32),
            pltpu.SemaphoreType.DMA,
            pltpu.SemaphoreType.DMA,
        ],
    )
    def k(xf_hbm, tt_hbm, tail_hbm, biasf_hbm, out_hbm,
          slab, vm_c, vm_gidx, vm_flat, vm_blk, vm_bias,
          sh_slab, sem_s, sem_g):
        cid = lax.axis_index("c")
        sid = lax.axis_index("s")
        c0 = sid * CW

        pltpu.sync_copy(
            biasf_hbm.at[pl.ds(cid * FPC * D * _L, FPC * D * _L)], vm_bias)

        def stage_copies(f, t):
            main = (tt_hbm.at[f, pl.ds(8 * t, 8), pl.ds(c0, CW)],
                    slab.at[:, pl.ds(0, CW)])
            last = [(tt_hbm.at[f, pl.ds(8 * t, 8), pl.ds(c0, CW_LAST)],
                     slab.at[:, pl.ds(0, CW_LAST)])]
            if C_TAIL:
                last.append((tail_hbm.at[f, pl.ds(8 * t, 8)],
                             slab.at[:, pl.ds(CW_LAST, 128)]))
            return main, last

        def fire_stage(f, t):
            main, last = stage_copies(f, t)

            @pl.when(sid < NS - 1)
            def _():
                pltpu.async_copy(*main, sem_s)

            @pl.when(sid == NS - 1)
            def _():
                for src, dst in last:
                    pltpu.async_copy(src, dst, sem_s)

        def wait_stage(f, t):
            main, last = stage_copies(f, t)

            @pl.when(sid < NS - 1)
            def _():
                pltpu.make_async_copy(*main, sem_s).wait()

            @pl.when(sid == NS - 1)
            def _():
                for src, dst in last:
                    pltpu.make_async_copy(src, dst, sem_s).wait()

        fire_stage(cid * FPC, 0)

        def feature_body(fi, _):
            f = cid * FPC + fi
            pltpu.sync_copy(xf_hbm.at[pl.ds(f * B + sid * BW, BW)], vm_c)

            def gidx_body(v, _):
                cvec = vm_c[pl.ds(v * _L, _L)]
                for r in range(4):
                    vm_gidx[pl.ds(r * BW + v * _L, _L)] = cvec + r * SW
                return _
            lax.fori_loop(0, BW // _L, gidx_body, None)

            for t in range(NT):
                for h in range(2):
                    if h == 0:
                        wait_stage(f, t)
                    for r in range(4):
                        @pl.when(sid < NS - 1)
                        def _(r=r, h=h):
                            pltpu.sync_copy(
                                slab.at[4 * h + r, pl.ds(0, CW)],
                                sh_slab.at[pl.ds(r * SW + c0, CW)])

                        @pl.when(sid == NS - 1)
                        def _(r=r, h=h):
                            pltpu.sync_copy(
                                slab.at[4 * h + r, pl.ds(0, WL)],
                                sh_slab.at[pl.ds(r * SW + c0, WL)])
                    if h == 1:
                        if t < NT - 1:
                            fire_stage(f, t + 1)
                        else:
                            @pl.when(fi < FPC - 1)
                            def _():
                                fire_stage(f + 1, 0)
                    plsc.subcore_barrier()

                    pltpu.async_copy(sh_slab.at[vm_gidx], vm_flat, sem_g)

                    pltpu.make_async_copy(
                        sh_slab.at[vm_gidx], vm_flat, sem_g).wait()

                    for r in range(4):
                        bvec = vm_bias[
                            pl.ds((fi * D + 8 * t + 4 * h + r) * _L, _L)]

                        def row_body(v, _, r=r, h=h, bvec=bvec):
                            sl = pl.ds(v * _L, _L)
                            vm_blk[4 * h + r, sl] = (
                                vm_flat[pl.ds(r * BW + v * _L, _L)] + bvec)
                            return _
                        lax.fori_loop(0, BW // _L, row_body, None)
                    plsc.subcore_barrier()

                pltpu.sync_copy(
                    vm_blk,
                    out_hbm.at[f, pl.ds(8 * t, 8), pl.ds(sid * BW, BW)])
            return _

        lax.fori_loop(0, FPC, feature_body, None)

    return k, C_MAIN, C_TAIL


def kernel(x, tables, bias):
    B, NF = x.shape
    NF2, CARD, D = tables.shape
    assert NF2 == NF
    info = plsc.get_sparse_core_info()
    NC, NS = info.num_cores, info.num_subcores

    impl, C_MAIN, C_TAIL = _make_impl(B, NF, CARD, D, NC, NS)
    xf = x.astype(jnp.int32).T.reshape(-1)
    tt = jnp.transpose(tables, (0, 2, 1))
    tail = jnp.transpose(tables[:, C_MAIN:, :], (0, 2, 1)) if C_TAIL \
        else jnp.zeros((NF, D, 0), tables.dtype)
    tail = jnp.pad(tail, ((0, 0), (0, 0), (0, 128 - tail.shape[2])))
    biasf = jnp.repeat(bias.reshape(-1)[:, None], _L, axis=1).reshape(-1)
    out3 = impl(xf, tt, tail, biasf)
    return jnp.transpose(out3, (2, 0, 1))
